# Initial kernel scaffold; baseline (speedup 1.0000x reference)
import jax
import jax.numpy as jnp
from jax.experimental import pallas as pl


def kernel(boxes, scores):
    raise NotImplementedError("write your pallas kernel here")


# speedup vs baseline: 21.2695x; 21.2695x over previous
import functools

import jax
import jax.numpy as jnp
from jax import lax
from jax.experimental import pallas as pl
from jax.experimental.pallas import tpu as pltpu
from jax.experimental.pallas import tpu_sc as plsc

_N = 20000
_K = 200
_T = 0.6
_L = 16
_KPAD = 13 * _L
_OUT_PAD = 1024
_B = 16


def _nms_body(x1_h, y1_h, x2_h, y2_h, ss_h, ord_h, init_h, out_h,
              x1_v, y1_v, x2_v, y2_v, ss_v, ord_v, out_v,
              kx1, ky1, kx2, ky2, kar, nk_ref):
    cid = lax.axis_index("c")
    sid = lax.axis_index("s")
    is0 = jnp.logical_and(cid == 0, sid == 0)

    pltpu.sync_copy(x1_h, x1_v)
    pltpu.sync_copy(y1_h, y1_v)
    pltpu.sync_copy(x2_h, x2_v)
    pltpu.sync_copy(y2_h, y2_v)
    pltpu.sync_copy(ss_h, ss_v)
    pltpu.sync_copy(ord_h, ord_v)
    pltpu.sync_copy(init_h, out_v)

    lanes = lax.iota(jnp.int32, _L)
    sent = jnp.full((_L,), -1e6, jnp.float32)
    zero = jnp.zeros((_L,), jnp.float32)

    def init_chunk(c, carry):
        s = pl.ds(c * _L, _L)
        kx1[s] = sent
        ky1[s] = sent
        kx2[s] = sent
        ky2[s] = sent
        kar[s] = zero
        return carry

    lax.fori_loop(0, _KPAD // _L, init_chunk, 0)
    nk_ref[0] = jnp.int32(0)

    def cand(j, carry):
        p, _ = carry
        nk = nk_ref[0]
        pv = jnp.full((_L,), p, jnp.int32)
        opv = plsc.load_gather(ord_v, [pv])
        cx1 = plsc.load_gather(x1_v, [opv])
        cy1 = plsc.load_gather(y1_v, [opv])
        cx2 = plsc.load_gather(x2_v, [opv])
        cy2 = plsc.load_gather(y2_v, [opv])
        cs = plsc.load_gather(ss_v, [opv])
        car = (cx2 - cx1) * (cy2 - cy1)

        def chunk(c, acc):
            s = pl.ds(c * _L, _L)
            ltx = jnp.maximum(cx1, kx1[s])
            lty = jnp.maximum(cy1, ky1[s])
            rbx = jnp.minimum(cx2, kx2[s])
            rby = jnp.minimum(cy2, ky2[s])
            w = jnp.maximum(rbx - ltx, 0.0)
            h = jnp.maximum(rby - lty, 0.0)
            inter = w * h
            union = car + kar[s] - inter + 1e-9
            return jnp.maximum(acc, inter / union)

        nchunks = (nk + _L - 1) // _L
        miou = lax.fori_loop(0, nchunks, chunk,
                             jnp.zeros((_L,), jnp.float32))
        keep = jnp.logical_and(jnp.max(miou) <= _T, nk < _K)

        keep_v = jnp.full((_L,), keep)
        nkv = jnp.full((_L,), nk, jnp.int32)
        m0 = jnp.logical_and(lanes == 0, keep_v)
        plsc.store_scatter(kx1, [nkv], cx1, mask=m0)
        plsc.store_scatter(ky1, [nkv], cy1, mask=m0)
        plsc.store_scatter(kx2, [nkv], cx2, mask=m0)
        plsc.store_scatter(ky2, [nkv], cy2, mask=m0)
        plsc.store_scatter(kar, [nkv], car, mask=m0)
        vals = jnp.where(lanes == 0, cx1,
                         jnp.where(lanes == 1, cy1,
                                   jnp.where(lanes == 2, cx2,
                                             jnp.where(lanes == 3,
                                                       cy2, cs))))
        plsc.store_scatter(out_v, [nk * 5 + lanes], vals,
                           mask=jnp.logical_and(lanes < 5, keep_v))

        nk_ref[0] = jnp.where(keep, nk + 1, nk)
        return p + 1, 0

    def blk(b, carry):
        @pl.when(nk_ref[0] < _K)
        def _():
            lax.fori_loop(0, _B, cand, (b * _B, 0))
        return carry

    lax.fori_loop(0, _N // _B, blk, 0)

    @pl.when(is0)
    def _():
        pltpu.sync_copy(out_v, out_h)


_nms = pl.kernel(
    _nms_body,
    out_type=jax.ShapeDtypeStruct((_OUT_PAD,), jnp.float32),
    mesh=plsc.VectorSubcoreMesh(core_axis_name="c", subcore_axis_name="s",
                                num_cores=2, num_subcores=16),
    compiler_params=pltpu.CompilerParams(needs_layout_passes=False),
    scratch_types=[
        pltpu.VMEM((_N,), jnp.float32),
        pltpu.VMEM((_N,), jnp.float32),
        pltpu.VMEM((_N,), jnp.float32),
        pltpu.VMEM((_N,), jnp.float32),
        pltpu.VMEM((_N,), jnp.float32),
        pltpu.VMEM((_N,), jnp.int32),
        pltpu.VMEM((_OUT_PAD,), jnp.float32),
        pltpu.VMEM((_KPAD,), jnp.float32),
        pltpu.VMEM((_KPAD,), jnp.float32),
        pltpu.VMEM((_KPAD,), jnp.float32),
        pltpu.VMEM((_KPAD,), jnp.float32),
        pltpu.VMEM((_KPAD,), jnp.float32),
        pltpu.SMEM((1,), jnp.int32),
    ],
)


def kernel(boxes, scores):
    order = jnp.argsort(-scores, stable=True).astype(jnp.int32)
    pad_row = jnp.concatenate(
        [boxes[0], jnp.full((1,), -1.0, jnp.float32)])
    init = jnp.resize(pad_row, (_OUT_PAD,))
    out = _nms(boxes[:, 0], boxes[:, 1], boxes[:, 2], boxes[:, 3],
               scores, order, init)
    return out[: _K * 5].reshape(_K, 5)
